# baseline (speedup 1.0000x reference)
import functools

import jax
import jax.numpy as jnp
from jax import lax
from jax.experimental import pallas as pl
from jax.experimental.pallas import tpu as pltpu
from jax.experimental.pallas import tpu_sc as plsc

N_NODES = 10000
N_EDGES = 320000
EMB_DIM = 128
MAX_DEG = 256

NC = 2
NS = 16
NW = NC * NS
EDGES_PER_W = N_EDGES // NW

BINS = 10240
NODE_BLK = 1024
TC_GRID = (N_NODES + NODE_BLK - 1) // NODE_BLK


def _i(v):
    return jnp.int32(v)


def _sc_degree_body(edges_hbm, out_hbm, src_v, dst_v,
                    hin_v, hout_v, hin2_v, hout2_v, sem):
    c = lax.axis_index("c")
    s = lax.axis_index("s")
    w = c * _i(NS) + s
    base = w * _i(EDGES_PER_W)

    cp_s = pltpu.async_copy(edges_hbm.at[pl.ds(base, EDGES_PER_W)], src_v, sem)
    cp_d = pltpu.async_copy(
        edges_hbm.at[pl.ds(base + _i(N_EDGES), EDGES_PER_W)], dst_v, sem)

    zeros = jnp.zeros((16,), jnp.int32)

    @plsc.parallel_loop(_i(0), _i(BINS // 32), _i(1), unroll=2)
    def _(i):
        o = i * _i(32)
        for hv in (hin_v, hout_v, hin2_v, hout2_v):
            hv[pl.ds(o, 16)] = zeros
            hv[pl.ds(o + _i(16), 16)] = zeros

    cp_s.wait()
    cp_d.wait()

    ones = jnp.ones((16,), jnp.int32)

    @plsc.parallel_loop(_i(0), _i(EDGES_PER_W // 32), _i(1), unroll=2)
    def _(i):
        o2 = i * _i(32)
        for k, (hi, ho) in enumerate(((hin_v, hout_v), (hin2_v, hout2_v))):
            ok = o2 + _i(16 * k)
            sv = src_v[pl.ds(ok, 16)]
            dv = dst_v[pl.ds(ok, 16)]
            m = sv != dv
            plsc.addupdate_scatter(ho, [sv], ones, mask=m)
            plsc.addupdate_scatter(hi, [dv], ones, mask=m)

    for r in range(EDGES_PER_W - EDGES_PER_W % 32, EDGES_PER_W, 16):
        sv = src_v[pl.ds(_i(r), 16)]
        dv = dst_v[pl.ds(_i(r), 16)]
        m = sv != dv
        plsc.addupdate_scatter(hout_v, [sv], ones, mask=m)
        plsc.addupdate_scatter(hin_v, [dv], ones, mask=m)

    pltpu.sync_copy(hin_v, out_hbm.at[w])
    pltpu.sync_copy(hin2_v, out_hbm.at[_i(NW) + w])
    pltpu.sync_copy(hout_v, out_hbm.at[_i(2 * NW) + w])
    pltpu.sync_copy(hout2_v, out_hbm.at[_i(3 * NW) + w])


@functools.cache
def _get_sc_degree():
    return functools.partial(
        pl.kernel,
        out_type=jax.ShapeDtypeStruct((4 * NW, BINS), jnp.int32),
        mesh=plsc.VectorSubcoreMesh(core_axis_name="c", subcore_axis_name="s"),
        compiler_params=pltpu.CompilerParams(needs_layout_passes=False),
        scratch_types=[
            pltpu.VMEM((EDGES_PER_W,), jnp.int32),
            pltpu.VMEM((EDGES_PER_W,), jnp.int32),
            pltpu.VMEM((BINS,), jnp.int32),
            pltpu.VMEM((BINS,), jnp.int32),
            pltpu.VMEM((BINS,), jnp.int32),
            pltpu.VMEM((BINS,), jnp.int32),
            pltpu.SemaphoreType.DMA,
        ],
    )(_sc_degree_body)


def _tc_lookup_body(x_ref, deg_ref, win_ref, wout_ref, o_ref):
    part = deg_ref[...]
    din = jnp.minimum(
        jnp.sum(part[:2 * NW], axis=0, keepdims=True, dtype=jnp.int32),
        MAX_DEG - 1)
    dout = jnp.minimum(
        jnp.sum(part[2 * NW:], axis=0, keepdims=True, dtype=jnp.int32),
        MAX_DEG - 1)
    iot = lax.broadcasted_iota(jnp.int32, (MAX_DEG, NODE_BLK), 0)
    oh_in = (jnp.broadcast_to(din, (MAX_DEG, NODE_BLK)) == iot).astype(jnp.bfloat16)
    oh_out = (jnp.broadcast_to(dout, (MAX_DEG, NODE_BLK)) == iot).astype(jnp.bfloat16)
    dn = (((0,), (0,)), ((), ()))

    def _split2(t):
        t1 = t.astype(jnp.bfloat16)
        t2 = (t - t1.astype(jnp.float32)).astype(jnp.bfloat16)
        return (t1, t2)

    acc = x_ref[...]
    for oh, tref in ((oh_in, win_ref), (oh_out, wout_ref)):
        for tk in _split2(tref[...]):
            acc = acc + lax.dot_general(oh, tk, dn,
                                        preferred_element_type=jnp.float32)
    o_ref[...] = acc


def _tc_lookup(x, partial, w_in, w_out):
    return pl.pallas_call(
        _tc_lookup_body,
        grid=(TC_GRID,),
        in_specs=[
            pl.BlockSpec((NODE_BLK, EMB_DIM), lambda i: (i, _i(0))),
            pl.BlockSpec((4 * NW, NODE_BLK), lambda i: (_i(0), i)),
            pl.BlockSpec((MAX_DEG, EMB_DIM), lambda i: (_i(0), _i(0))),
            pl.BlockSpec((MAX_DEG, EMB_DIM), lambda i: (_i(0), _i(0))),
        ],
        out_specs=pl.BlockSpec((NODE_BLK, EMB_DIM), lambda i: (i, _i(0))),
        out_shape=jax.ShapeDtypeStruct((N_NODES, EMB_DIM), jnp.float32),
    )(x, partial, w_in, w_out)


def kernel(x, edge_index, in_emb_weight, out_emb_weight):
    e_flat = edge_index.astype(jnp.int32).reshape(2 * N_EDGES)
    partial = _get_sc_degree()(e_flat)
    return _tc_lookup(x, partial, in_emb_weight, out_emb_weight)


# speedup vs baseline: 1.0761x; 1.0761x over previous
import functools

import jax
import jax.numpy as jnp
from jax import lax
from jax.experimental import pallas as pl
from jax.experimental.pallas import tpu as pltpu
from jax.experimental.pallas import tpu_sc as plsc

N_NODES = 10000
N_EDGES = 320000
EMB_DIM = 128
MAX_DEG = 256

NC = 2
NS = 16
NW = NC * NS
EDGES_PER_W = N_EDGES // NW

BINS = 10240
NODE_BLK = 2560
TC_GRID = (N_NODES + NODE_BLK - 1) // NODE_BLK


def _i(v):
    return jnp.int32(v)


def _sc_degree_body(edges_hbm, out_hbm, src_v, dst_v,
                    hin_v, hout_v, hin2_v, hout2_v, sem):
    c = lax.axis_index("c")
    s = lax.axis_index("s")
    w = c * _i(NS) + s
    base = w * _i(EDGES_PER_W)

    cp_s = pltpu.async_copy(edges_hbm.at[pl.ds(base, EDGES_PER_W)], src_v, sem)
    cp_d = pltpu.async_copy(
        edges_hbm.at[pl.ds(base + _i(N_EDGES), EDGES_PER_W)], dst_v, sem)

    zeros = jnp.zeros((16,), jnp.int32)

    @plsc.parallel_loop(_i(0), _i(BINS // 32), _i(1), unroll=2)
    def _(i):
        o = i * _i(32)
        for hv in (hin_v, hout_v, hin2_v, hout2_v):
            hv[pl.ds(o, 16)] = zeros
            hv[pl.ds(o + _i(16), 16)] = zeros

    cp_s.wait()
    cp_d.wait()

    ones = jnp.ones((16,), jnp.int32)

    @plsc.parallel_loop(_i(0), _i(EDGES_PER_W // 32), _i(1), unroll=2)
    def _(i):
        o2 = i * _i(32)
        for k, (hi, ho) in enumerate(((hin_v, hout_v), (hin2_v, hout2_v))):
            ok = o2 + _i(16 * k)
            sv = src_v[pl.ds(ok, 16)]
            dv = dst_v[pl.ds(ok, 16)]
            m = sv != dv
            plsc.addupdate_scatter(ho, [sv], ones, mask=m)
            plsc.addupdate_scatter(hi, [dv], ones, mask=m)

    for r in range(EDGES_PER_W - EDGES_PER_W % 32, EDGES_PER_W, 16):
        sv = src_v[pl.ds(_i(r), 16)]
        dv = dst_v[pl.ds(_i(r), 16)]
        m = sv != dv
        plsc.addupdate_scatter(hout_v, [sv], ones, mask=m)
        plsc.addupdate_scatter(hin_v, [dv], ones, mask=m)

    pltpu.sync_copy(hin_v, out_hbm.at[w])
    pltpu.sync_copy(hin2_v, out_hbm.at[_i(NW) + w])
    pltpu.sync_copy(hout_v, out_hbm.at[_i(2 * NW) + w])
    pltpu.sync_copy(hout2_v, out_hbm.at[_i(3 * NW) + w])


@functools.cache
def _get_sc_degree():
    return functools.partial(
        pl.kernel,
        out_type=jax.ShapeDtypeStruct((4 * NW, BINS), jnp.int32),
        mesh=plsc.VectorSubcoreMesh(core_axis_name="c", subcore_axis_name="s"),
        compiler_params=pltpu.CompilerParams(needs_layout_passes=False),
        scratch_types=[
            pltpu.VMEM((EDGES_PER_W,), jnp.int32),
            pltpu.VMEM((EDGES_PER_W,), jnp.int32),
            pltpu.VMEM((BINS,), jnp.int32),
            pltpu.VMEM((BINS,), jnp.int32),
            pltpu.VMEM((BINS,), jnp.int32),
            pltpu.VMEM((BINS,), jnp.int32),
            pltpu.SemaphoreType.DMA,
        ],
    )(_sc_degree_body)


def _tc_lookup_body(x_ref, deg_ref, win_ref, wout_ref, o_ref):
    part = deg_ref[...]
    din = jnp.minimum(
        jnp.sum(part[:2 * NW], axis=0, keepdims=True, dtype=jnp.int32),
        MAX_DEG - 1)
    dout = jnp.minimum(
        jnp.sum(part[2 * NW:], axis=0, keepdims=True, dtype=jnp.int32),
        MAX_DEG - 1)
    iot = lax.broadcasted_iota(jnp.int32, (MAX_DEG, NODE_BLK), 0)
    oh_in = (jnp.broadcast_to(din, (MAX_DEG, NODE_BLK)) == iot).astype(jnp.bfloat16)
    oh_out = (jnp.broadcast_to(dout, (MAX_DEG, NODE_BLK)) == iot).astype(jnp.bfloat16)
    dn = (((0,), (0,)), ((), ()))

    def _split2(t):
        t1 = t.astype(jnp.bfloat16)
        t2 = (t - t1.astype(jnp.float32)).astype(jnp.bfloat16)
        return (t1, t2)

    acc = x_ref[...]
    for oh, tref in ((oh_in, win_ref), (oh_out, wout_ref)):
        for tk in _split2(tref[...]):
            acc = acc + lax.dot_general(oh, tk, dn,
                                        preferred_element_type=jnp.float32)
    o_ref[...] = acc


def _tc_lookup(x, partial, w_in, w_out):
    return pl.pallas_call(
        _tc_lookup_body,
        grid=(TC_GRID,),
        in_specs=[
            pl.BlockSpec((NODE_BLK, EMB_DIM), lambda i: (i, _i(0))),
            pl.BlockSpec((4 * NW, NODE_BLK), lambda i: (_i(0), i)),
            pl.BlockSpec((MAX_DEG, EMB_DIM), lambda i: (_i(0), _i(0))),
            pl.BlockSpec((MAX_DEG, EMB_DIM), lambda i: (_i(0), _i(0))),
        ],
        out_specs=pl.BlockSpec((NODE_BLK, EMB_DIM), lambda i: (i, _i(0))),
        out_shape=jax.ShapeDtypeStruct((N_NODES, EMB_DIM), jnp.float32),
    )(x, partial, w_in, w_out)


def kernel(x, edge_index, in_emb_weight, out_emb_weight):
    e_flat = edge_index.astype(jnp.int32).reshape(2 * N_EDGES)
    partial = _get_sc_degree()(e_flat)
    return _tc_lookup(x, partial, in_emb_weight, out_emb_weight)
